# baseline (speedup 1.0000x reference)
import jax
import jax.numpy as jnp
import numpy as np
from jax import lax
from jax.experimental import pallas as pl
from jax.experimental.pallas import tpu as pltpu
from jax.experimental.pallas import tpu_sc as plsc

N = 10000
E = 320000
D = 128
DE = 16
L = 3

NC = 2
NS = 16
NW = NC * NS
EPW = E // NW
CHUNK = 40
NCHUNK = EPW // CHUNK
NBUF = 2
NIB = 10
CP = 40
NCP = N // CP
CP_ITERS = -(-NCP // NS)


def _sc_body(h_hbm, emb_hbm, src_hbm, dst_hbm, out_hbm, *refs):
    srcv = refs[0:NIB]
    dstv = refs[NIB:2 * NIB]
    rows, embv, sctv, zbuf, agg_sh, isem, lsem, ssem, zsem = refs[2 * NIB:]

    c = lax.axis_index("c")
    s = lax.axis_index("s")
    wid = s * NC + c
    e0 = wid * EPW

    def issue_idx(g, sl):
        base = e0 + g * CHUNK
        pltpu.async_copy(src_hbm.at[pl.ds(base, CHUNK)], srcv[sl],
                         isem.at[sl])
        pltpu.async_copy(dst_hbm.at[pl.ds(base, CHUNK)], dstv[sl],
                         isem.at[sl])

    def wait_idx(sl):
        pltpu.make_async_copy(src_hbm.at[pl.ds(0, CHUNK)], srcv[sl],
                              isem.at[sl]).wait()
        pltpu.make_async_copy(src_hbm.at[pl.ds(0, CHUNK)], dstv[sl],
                              isem.at[sl]).wait()

    def issue_loads(g, sl, db):
        pltpu.async_copy(emb_hbm.at[pl.ds(e0 + g * CHUNK, CHUNK)],
                         embv.at[db], lsem.at[db])
        pltpu.async_copy(h_hbm.at[srcv[sl]], rows.at[db], lsem.at[db])

    def wait_loads(db):
        pltpu.make_async_copy(emb_hbm.at[pl.ds(0, CHUNK)], embv.at[db],
                              lsem.at[db]).wait()
        pltpu.make_async_copy(emb_hbm.at[pl.ds(0, CHUNK)], rows.at[db],
                              lsem.at[db]).wait()

    def wait_scatter(db):
        pltpu.make_async_copy(emb_hbm.at[pl.ds(0, CHUNK)], sctv.at[db],
                              ssem.at[db]).wait()

    for g in range(4):
        issue_idx(g, g)
    for g in range(NBUF):
        wait_idx(g)
        issue_loads(g, g, g)

    @pl.loop(0, CP)
    def _zero(i):
        for j in range(D // 16):
            zbuf[i, pl.ds(j * 16, 16)] = jnp.zeros((16,), jnp.float32)

    @pl.loop(0, CP_ITERS)
    def _zc(i):
        idx = s + i * NS

        @pl.when(idx < NCP)
        def _():
            pltpu.async_copy(zbuf, agg_sh.at[pl.ds(idx * CP, CP)], zsem)

    @pl.loop(0, CP_ITERS)
    def _zw(i):
        idx = s + i * NS

        @pl.when(idx < NCP)
        def _():
            pltpu.make_async_copy(zbuf, agg_sh.at[pl.ds(0, CP)],
                                  zsem).wait()

    plsc.subcore_barrier()

    @pl.loop(0, NCHUNK, step=NIB)
    def _ring(g0):
        for b in range(NIB):
            g = g0 + b
            db = b % NBUF
            wait_loads(db)

            @pl.when(g >= NBUF)
            def _():
                wait_scatter(db)

            @plsc.parallel_loop(0, CHUNK, unroll=2)
            def _edge(e):
                for k in range(D // 32):
                    iw = embv[db, e, pl.ds(k * 16, 16)]
                    ea = lax.bitcast_convert_type(
                        lax.shift_left(iw, 16), jnp.float32)
                    eb = lax.bitcast_convert_type(
                        lax.bitwise_and(iw, jnp.int32(-65536)), jnp.float32)
                    s0 = (db, e, pl.ds(k * 16, 16))
                    s1 = (db, e, pl.ds(D // 2 + k * 16, 16))
                    sctv[s0] = jnp.maximum(rows[s0] + ea, 0.0)
                    sctv[s1] = jnp.maximum(rows[s1] + eb, 0.0)

            pltpu.async_copy(sctv.at[db], agg_sh.at[dstv[b]], ssem.at[db],
                             add=True)

            n = g + NBUF

            @pl.when(n < NCHUNK)
            def _():
                wait_idx((b + NBUF) % NIB)
                issue_loads(n, (b + NBUF) % NIB, db)

            m = g + 4

            @pl.when(m < NCHUNK)
            def _():
                issue_idx(m, (b + 4) % NIB)

    for db in range(NBUF):
        wait_scatter(db)
    plsc.subcore_barrier()

    @pl.loop(0, CP_ITERS)
    def _dump(i):
        idx = s + i * NS

        @pl.when(idx < NCP)
        def _():
            pltpu.async_copy(agg_sh.at[pl.ds(idx * CP, CP)],
                             out_hbm.at[c, pl.ds(idx * CP, CP)], zsem)

    @pl.loop(0, CP_ITERS)
    def _dw(i):
        idx = s + i * NS

        @pl.when(idx < NCP)
        def _():
            pltpu.make_async_copy(agg_sh.at[pl.ds(0, CP)],
                                  out_hbm.at[c, pl.ds(0, CP)], zsem).wait()


def _sc_aggregate(h, emb, src, dst):
    mesh = plsc.VectorSubcoreMesh(core_axis_name="c", subcore_axis_name="s",
                                  num_cores=NC, num_subcores=NS)
    f = pl.kernel(
        _sc_body,
        out_type=jax.ShapeDtypeStruct((NC, N, D), jnp.float32),
        mesh=mesh,
        scratch_types=(
            [pltpu.VMEM((CHUNK,), jnp.int32)] * (2 * NIB)
            + [
                pltpu.VMEM((NBUF, CHUNK, D), jnp.float32),
                pltpu.VMEM((NBUF, CHUNK, D // 2), jnp.int32),
                pltpu.VMEM((NBUF, CHUNK, D), jnp.float32),
                pltpu.VMEM((CP, D), jnp.float32),
                pltpu.VMEM_SHARED((N, D), jnp.float32),
                pltpu.SemaphoreType.DMA((NIB,)),
                pltpu.SemaphoreType.DMA((NBUF,)),
                pltpu.SemaphoreType.DMA((NBUF,)),
                pltpu.SemaphoreType.DMA,
            ]
        ),
    )
    return f(h, emb, src, dst)


def _mm_bias_body(x_ref, w_ref, b_ref, o_ref):
    o_ref[...] = jnp.dot(x_ref[...], w_ref[...],
                         preferred_element_type=jnp.float32) + b_ref[...]


def _mm_bias(x, w, b, bn):
    n = x.shape[0]
    k = x.shape[1]
    d = w.shape[1]
    return pl.pallas_call(
        _mm_bias_body,
        grid=(n // bn,),
        in_specs=[
            pl.BlockSpec((bn, k), lambda i: (i, 0)),
            pl.BlockSpec((k, d), lambda i: (0, 0)),
            pl.BlockSpec((1, d), lambda i: (0, 0)),
        ],
        out_specs=pl.BlockSpec((bn, d), lambda i: (i, 0)),
        out_shape=jax.ShapeDtypeStruct((n, d), jnp.float32),
    )(x, w, b.reshape(1, d))


def _emb_body(ea_ref, we_ref, be_ref, o_ref):
    z = jnp.dot(ea_ref[...], we_ref[...],
                preferred_element_type=jnp.float32) + be_ref[...]
    u = lax.bitcast_convert_type(z, jnp.uint32)
    r = u + jnp.uint32(0x8000)
    lo = r[:, : D // 2] >> 16
    hi = r[:, D // 2:] & jnp.uint32(0xFFFF0000)
    o_ref[...] = lax.bitcast_convert_type(lo | hi, jnp.int32)


def _emb_one(edge_attr, We_l, be_l, eb=4000):
    return pl.pallas_call(
        _emb_body,
        grid=(E // eb,),
        in_specs=[
            pl.BlockSpec((eb, DE), lambda i: (i, 0)),
            pl.BlockSpec((DE, D), lambda i: (0, 0)),
            pl.BlockSpec((1, D), lambda i: (0, 0)),
        ],
        out_specs=pl.BlockSpec((eb, D // 2), lambda i: (i, 0)),
        out_shape=jax.ShapeDtypeStruct((E, D // 2), jnp.int32),
    )(edge_attr, We_l, be_l.reshape(1, D))


def _mlp_body(scale_ref, h_ref, p0_ref, p1_ref, w1_ref, b1_ref, w2_ref,
              b2_ref, g_ref, bt_ref, o_ref):
    z = scale_ref[...] * h_ref[...] + (p0_ref[...] + p1_ref[...])
    a = jnp.maximum(jnp.dot(z, w1_ref[...],
                            preferred_element_type=jnp.float32) + b1_ref[...], 0.0)
    z2 = jnp.dot(a, w2_ref[...],
                 preferred_element_type=jnp.float32) + b2_ref[...]
    mu = jnp.mean(z2, axis=-1, keepdims=True)
    var = jnp.mean((z2 - mu) * (z2 - mu), axis=-1, keepdims=True)
    o_ref[...] = g_ref[...] * (z2 - mu) * lax.rsqrt(var + 1e-5) + bt_ref[...]


def _mlp_last_body(scale_ref, h_ref, p0_ref, p1_ref, w1_ref, b1_ref, w2_ref,
                   b2_ref, g_ref, bt_ref, wjk_ref, bjk_ref, o_ref):
    z = scale_ref[...] * h_ref[...] + (p0_ref[...] + p1_ref[...])
    a = jnp.maximum(jnp.dot(z, w1_ref[...],
                            preferred_element_type=jnp.float32) + b1_ref[...], 0.0)
    z2 = jnp.dot(a, w2_ref[...],
                 preferred_element_type=jnp.float32) + b2_ref[...]
    mu = jnp.mean(z2, axis=-1, keepdims=True)
    var = jnp.mean((z2 - mu) * (z2 - mu), axis=-1, keepdims=True)
    hn = g_ref[...] * (z2 - mu) * lax.rsqrt(var + 1e-5) + bt_ref[...]
    o_ref[...] = jnp.dot(hn, wjk_ref[...],
                         preferred_element_type=jnp.float32) + bjk_ref[...]


def _mlp(scale, h, p0, p1, w1, b1, w2, b2, g, bt, wjk=None, bjk=None,
         bn=1000):
    wspec = pl.BlockSpec((D, D), lambda i: (0, 0))
    vspec = pl.BlockSpec((1, D), lambda i: (0, 0))
    hspec = pl.BlockSpec((bn, D), lambda i: (i, 0))
    args = [scale, h, p0, p1, w1, b1.reshape(1, D), w2, b2.reshape(1, D),
            g.reshape(1, D), bt.reshape(1, D)]
    specs = [vspec, hspec, hspec, hspec, wspec, vspec, wspec, vspec,
             vspec, vspec]
    body = _mlp_body
    if wjk is not None:
        args += [wjk, bjk.reshape(1, D)]
        specs += [wspec, vspec]
        body = _mlp_last_body
    return pl.pallas_call(
        body,
        grid=(N // bn,),
        in_specs=specs,
        out_specs=hspec,
        out_shape=jax.ShapeDtypeStruct((N, D), jnp.float32),
    )(*args)


@jax.jit
def kernel(x, edge_index, edge_attr, W_init, b_init, We, be, eps, W1, b1,
           W2, b2, gamma, beta, Wjk, bjk):
    src = edge_index[0]
    dst = edge_index[1]
    h = _mm_bias(x, W_init, b_init, bn=1000)
    embs = [_emb_one(edge_attr, We[l], be[l]) for l in range(L)]
    for l in range(L):
        parts = _sc_aggregate(h, embs[l], src, dst)
        scale = jnp.broadcast_to((1.0 + eps[l]).reshape(1, 1), (1, D))
        last = l == L - 1
        h = _mlp(scale, h, parts[0], parts[1], W1[l], b1[l], W2[l], b2[l],
                 gamma[l], beta[l],
                 wjk=Wjk if last else None, bjk=bjk if last else None)
    return h


# speedup vs baseline: 1.0330x; 1.0330x over previous
import jax
import jax.numpy as jnp
from jax import lax
from jax.experimental import pallas as pl
from jax.experimental.pallas import tpu as pltpu
from jax.experimental.pallas import tpu_sc as plsc

N = 10000
E = 320000
D = 128
DE = 16
L = 3

NC = 2
NS = 16
NW = NC * NS
EPW = E // NW
CHUNK = 40
NCHUNK = EPW // CHUNK
NBUF = 2
NIB = 10
CP = 40
NCP = N // CP
CP_ITERS = -(-NCP // NS)


def _sc_body(h_hbm, emb_hbm, src_hbm, dst_hbm, out_hbm, *refs):
    srcv = refs[0:NIB]
    dstv = refs[NIB:2 * NIB]
    rows, embv, sctv, zbuf, agg_sh, isem, lsem, ssem, zsem = refs[2 * NIB:]

    c = lax.axis_index("c")
    s = lax.axis_index("s")
    wid = s * NC + c
    e0 = wid * EPW

    def issue_idx(g, sl):
        base = e0 + g * CHUNK
        pltpu.async_copy(src_hbm.at[pl.ds(base, CHUNK)], srcv[sl],
                         isem.at[sl])
        pltpu.async_copy(dst_hbm.at[pl.ds(base, CHUNK)], dstv[sl],
                         isem.at[sl])

    def wait_idx(sl):
        pltpu.make_async_copy(src_hbm.at[pl.ds(0, CHUNK)], srcv[sl],
                              isem.at[sl]).wait()
        pltpu.make_async_copy(src_hbm.at[pl.ds(0, CHUNK)], dstv[sl],
                              isem.at[sl]).wait()

    def issue_loads(g, sl, db):
        pltpu.async_copy(emb_hbm.at[pl.ds(e0 + g * CHUNK, CHUNK)],
                         embv.at[db], lsem.at[db])
        pltpu.async_copy(h_hbm.at[srcv[sl]], rows.at[db], lsem.at[db])

    def wait_loads(db):
        pltpu.make_async_copy(emb_hbm.at[pl.ds(0, CHUNK)], embv.at[db],
                              lsem.at[db]).wait()
        pltpu.make_async_copy(emb_hbm.at[pl.ds(0, CHUNK)], rows.at[db],
                              lsem.at[db]).wait()

    def wait_scatter(db):
        pltpu.make_async_copy(emb_hbm.at[pl.ds(0, CHUNK)], sctv.at[db],
                              ssem.at[db]).wait()

    for g in range(4):
        issue_idx(g, g)
    for g in range(NBUF):
        wait_idx(g)
        issue_loads(g, g, g)

    @pl.loop(0, CP)
    def _zero(i):
        for j in range(D // 16):
            zbuf[i, pl.ds(j * 16, 16)] = jnp.zeros((16,), jnp.float32)

    @pl.loop(0, CP_ITERS)
    def _zc(i):
        idx = s + i * NS

        @pl.when(idx < NCP)
        def _():
            pltpu.async_copy(zbuf, agg_sh.at[pl.ds(idx * CP, CP)], zsem)

    @pl.loop(0, CP_ITERS)
    def _zw(i):
        idx = s + i * NS

        @pl.when(idx < NCP)
        def _():
            pltpu.make_async_copy(zbuf, agg_sh.at[pl.ds(0, CP)],
                                  zsem).wait()

    plsc.subcore_barrier()

    @pl.loop(0, NCHUNK, step=NIB)
    def _ring(g0):
        for b in range(NIB):
            g = g0 + b
            db = b % NBUF
            wait_loads(db)

            @pl.when(g >= NBUF)
            def _():
                wait_scatter(db)

            @plsc.parallel_loop(0, CHUNK, unroll=2)
            def _edge(e):
                for k in range(D // 32):
                    iw = embv[db, e, pl.ds(k * 16, 16)]
                    ea = lax.bitcast_convert_type(
                        lax.shift_left(iw, 16), jnp.float32)
                    eb = lax.bitcast_convert_type(
                        lax.bitwise_and(iw, jnp.int32(-65536)), jnp.float32)
                    s0 = (db, e, pl.ds(k * 16, 16))
                    s1 = (db, e, pl.ds(D // 2 + k * 16, 16))
                    sctv[s0] = jnp.maximum(rows[s0] + ea, 0.0)
                    sctv[s1] = jnp.maximum(rows[s1] + eb, 0.0)

            pltpu.async_copy(sctv.at[db], agg_sh.at[dstv[b]], ssem.at[db],
                             add=True)

            n = g + NBUF

            @pl.when(n < NCHUNK)
            def _():
                wait_idx((b + NBUF) % NIB)
                issue_loads(n, (b + NBUF) % NIB, db)

            m = g + 4

            @pl.when(m < NCHUNK)
            def _():
                issue_idx(m, (b + 4) % NIB)

    for db in range(NBUF):
        wait_scatter(db)
    plsc.subcore_barrier()

    @pl.loop(0, CP_ITERS)
    def _dump(i):
        idx = s + i * NS

        @pl.when(idx < NCP)
        def _():
            pltpu.async_copy(agg_sh.at[pl.ds(idx * CP, CP)],
                             out_hbm.at[c, pl.ds(idx * CP, CP)], zsem)

    @pl.loop(0, CP_ITERS)
    def _dw(i):
        idx = s + i * NS

        @pl.when(idx < NCP)
        def _():
            pltpu.make_async_copy(agg_sh.at[pl.ds(0, CP)],
                                  out_hbm.at[c, pl.ds(0, CP)], zsem).wait()


def _sc_aggregate(h, emb, src, dst):
    mesh = plsc.VectorSubcoreMesh(core_axis_name="c", subcore_axis_name="s",
                                  num_cores=NC, num_subcores=NS)
    f = pl.kernel(
        _sc_body,
        out_type=jax.ShapeDtypeStruct((NC, N, D), jnp.float32),
        mesh=mesh,
        scratch_types=(
            [pltpu.VMEM((CHUNK,), jnp.int32)] * (2 * NIB)
            + [
                pltpu.VMEM((NBUF, CHUNK, D), jnp.float32),
                pltpu.VMEM((NBUF, CHUNK, D // 2), jnp.int32),
                pltpu.VMEM((NBUF, CHUNK, D), jnp.float32),
                pltpu.VMEM((CP, D), jnp.float32),
                pltpu.VMEM_SHARED((N, D), jnp.float32),
                pltpu.SemaphoreType.DMA((NIB,)),
                pltpu.SemaphoreType.DMA((NBUF,)),
                pltpu.SemaphoreType.DMA((NBUF,)),
                pltpu.SemaphoreType.DMA,
            ]
        ),
    )
    return f(h, emb, src, dst)


def _mm_bias_body(x_ref, w_ref, b_ref, o_ref):
    o_ref[...] = jnp.dot(x_ref[...], w_ref[...],
                         preferred_element_type=jnp.float32) + b_ref[...]


def _mm_bias(x, w, b, bn):
    n = x.shape[0]
    k = x.shape[1]
    d = w.shape[1]
    return pl.pallas_call(
        _mm_bias_body,
        grid=(n // bn,),
        in_specs=[
            pl.BlockSpec((bn, k), lambda i: (i, 0)),
            pl.BlockSpec((k, d), lambda i: (0, 0)),
            pl.BlockSpec((1, d), lambda i: (0, 0)),
        ],
        out_specs=pl.BlockSpec((bn, d), lambda i: (i, 0)),
        out_shape=jax.ShapeDtypeStruct((n, d), jnp.float32),
    )(x, w, b.reshape(1, d))


def _emb_body(ea_ref, we_ref, be_ref, o0_ref, o1_ref, o2_ref):
    for l, o_ref in enumerate((o0_ref, o1_ref, o2_ref)):
        z = jnp.dot(ea_ref[...], we_ref[l],
                    preferred_element_type=jnp.float32) + be_ref[l]
        u = lax.bitcast_convert_type(z, jnp.uint32)
        r = u + jnp.uint32(0x8000)
        lo = r[:, : D // 2] >> 16
        hi = r[:, D // 2:] & jnp.uint32(0xFFFF0000)
        o_ref[...] = lax.bitcast_convert_type(lo | hi, jnp.int32)


def _emb_all(edge_attr, We, be, eb=4000):
    out = jax.ShapeDtypeStruct((E, D // 2), jnp.int32)
    spec = pl.BlockSpec((eb, D // 2), lambda i: (i, 0))
    return pl.pallas_call(
        _emb_body,
        grid=(E // eb,),
        in_specs=[
            pl.BlockSpec((eb, DE), lambda i: (i, 0)),
            pl.BlockSpec((L, DE, D), lambda i: (0, 0, 0)),
            pl.BlockSpec((L, 1, D), lambda i: (0, 0, 0)),
        ],
        out_specs=[spec, spec, spec],
        out_shape=[out, out, out],
    )(edge_attr, We, be.reshape(L, 1, D))


def _mlp_body(scale_ref, h_ref, p0_ref, p1_ref, w1_ref, b1_ref, w2_ref,
              b2_ref, g_ref, bt_ref, o_ref):
    z = scale_ref[...] * h_ref[...] + (p0_ref[...] + p1_ref[...])
    a = jnp.maximum(jnp.dot(z, w1_ref[...],
                            preferred_element_type=jnp.float32) + b1_ref[...], 0.0)
    z2 = jnp.dot(a, w2_ref[...],
                 preferred_element_type=jnp.float32) + b2_ref[...]
    mu = jnp.mean(z2, axis=-1, keepdims=True)
    var = jnp.mean((z2 - mu) * (z2 - mu), axis=-1, keepdims=True)
    o_ref[...] = g_ref[...] * (z2 - mu) * lax.rsqrt(var + 1e-5) + bt_ref[...]


def _mlp_last_body(scale_ref, h_ref, p0_ref, p1_ref, w1_ref, b1_ref, w2_ref,
                   b2_ref, g_ref, bt_ref, wjk_ref, bjk_ref, o_ref):
    z = scale_ref[...] * h_ref[...] + (p0_ref[...] + p1_ref[...])
    a = jnp.maximum(jnp.dot(z, w1_ref[...],
                            preferred_element_type=jnp.float32) + b1_ref[...], 0.0)
    z2 = jnp.dot(a, w2_ref[...],
                 preferred_element_type=jnp.float32) + b2_ref[...]
    mu = jnp.mean(z2, axis=-1, keepdims=True)
    var = jnp.mean((z2 - mu) * (z2 - mu), axis=-1, keepdims=True)
    hn = g_ref[...] * (z2 - mu) * lax.rsqrt(var + 1e-5) + bt_ref[...]
    o_ref[...] = jnp.dot(hn, wjk_ref[...],
                         preferred_element_type=jnp.float32) + bjk_ref[...]


def _mlp(scale, h, p0, p1, w1, b1, w2, b2, g, bt, wjk=None, bjk=None,
         bn=1000):
    wspec = pl.BlockSpec((D, D), lambda i: (0, 0))
    vspec = pl.BlockSpec((1, D), lambda i: (0, 0))
    hspec = pl.BlockSpec((bn, D), lambda i: (i, 0))
    args = [scale, h, p0, p1, w1, b1.reshape(1, D), w2, b2.reshape(1, D),
            g.reshape(1, D), bt.reshape(1, D)]
    specs = [vspec, hspec, hspec, hspec, wspec, vspec, wspec, vspec,
             vspec, vspec]
    body = _mlp_body
    if wjk is not None:
        args += [wjk, bjk.reshape(1, D)]
        specs += [wspec, vspec]
        body = _mlp_last_body
    return pl.pallas_call(
        body,
        grid=(N // bn,),
        in_specs=specs,
        out_specs=hspec,
        out_shape=jax.ShapeDtypeStruct((N, D), jnp.float32),
    )(*args)


@jax.jit
def kernel(x, edge_index, edge_attr, W_init, b_init, We, be, eps, W1, b1,
           W2, b2, gamma, beta, Wjk, bjk):
    src = edge_index[0]
    dst = edge_index[1]
    h = _mm_bias(x, W_init, b_init, bn=1000)
    embs = _emb_all(edge_attr, We, be)
    for l in range(L):
        parts = _sc_aggregate(h, embs[l], src, dst)
        scale = jnp.broadcast_to((1.0 + eps[l]).reshape(1, 1), (1, D))
        last = l == L - 1
        h = _mlp(scale, h, parts[0], parts[1], W1[l], b1[l], W2[l], b2[l],
                 gamma[l], beta[l],
                 wjk=Wjk if last else None, bjk=bjk if last else None)
    return h
